# baseline (speedup 1.0000x reference)
import functools

import jax
import jax.numpy as jnp
from jax import lax
from jax.experimental import pallas as pl
from jax.experimental.pallas import tpu as pltpu
from jax.experimental.pallas import tpu_sc as plsc

VOCAB = 1000000
D = 64
VB = 8192
NBLK = -(-VOCAB // VB)
SUBS = 8
P = NBLK * VB

B = 4096
H = 200
HP = 256
NC = 2
NS = 16
NW = NC * NS
RPW = B // NW
K_ROWS = 4
NGRP = RPW // K_ROWS


def _proj_body(x_ref, w_ref, b_ref, o_ref):
    parts = []
    for s in range(SUBS):
        xs = x_ref[s * 1024:(s + 1) * 1024, :]
        parts.append(
            lax.dot_general(w_ref[...], xs, (((1,), (1,)), ((), ())),
                            preferred_element_type=jnp.float32))
    o_ref[0] = jnp.concatenate(parts, axis=0) + b_ref[0]


def _gather_body(proj_hbm, idx_hbm, out_hbm, idx_v, val_v, sem):
    wid = lax.axis_index("s") * NC + lax.axis_index("c")
    r0 = wid * RPW
    pltpu.sync_copy(idx_hbm.at[pl.ds(r0, RPW)], idx_v)

    def group(g, carry):
        base = g * K_ROWS
        copies = []
        for k in range(K_ROWS):
            r = base + k
            copies.append(pltpu.async_copy(
                proj_hbm.at[idx_v.at[r, pl.ds(0, 128)]],
                val_v.at[r, pl.ds(0, 128)], sem))
            copies.append(pltpu.async_copy(
                proj_hbm.at[idx_v.at[r, pl.ds(128, H - 128)]],
                val_v.at[r, pl.ds(128, H - 128)], sem))
        for c in copies:
            c.wait()
        return carry

    lax.fori_loop(0, NGRP, group, 0)
    pltpu.sync_copy(val_v, out_hbm.at[pl.ds(r0, RPW)])


def kernel(feature, lan_emb, W_w, b_w, W_out, b_out):
    proj3 = pl.pallas_call(
        _proj_body,
        grid=(NBLK,),
        in_specs=[
            pl.BlockSpec((VB, D), lambda i: (i, 0)),
            pl.BlockSpec((1, D), lambda i: (0, 0)),
            pl.BlockSpec(memory_space=pltpu.SMEM),
        ],
        out_specs=pl.BlockSpec((1, SUBS, 1024), lambda i: (i, 0, 0)),
        out_shape=jax.ShapeDtypeStruct((NBLK, SUBS, 1024), jnp.float32),
    )(lan_emb, W_out, b_out)
    proj = proj3.reshape(P)

    idx_pad = jnp.pad(feature.astype(jnp.int32), ((0, 0), (0, HP - H)))

    gather = functools.partial(
        pl.kernel,
        mesh=plsc.VectorSubcoreMesh(core_axis_name="c", subcore_axis_name="s"),
        out_type=jax.ShapeDtypeStruct((B, HP), jnp.float32),
        scratch_types=[
            pltpu.VMEM((RPW, HP), jnp.int32),
            pltpu.VMEM((RPW, HP), jnp.float32),
            pltpu.SemaphoreType.DMA,
        ],
    )(_gather_body)
    out_pad = gather(proj, idx_pad)

    return out_pad[:, :H]


# speedup vs baseline: 1.0120x; 1.0120x over previous
import functools

import jax
import jax.numpy as jnp
from jax import lax
from jax.experimental import pallas as pl
from jax.experimental.pallas import tpu as pltpu
from jax.experimental.pallas import tpu_sc as plsc

VOCAB = 1000000
D = 64
VB = 8192
NBLK = -(-VOCAB // VB)
SUBS = 8
P = NBLK * VB

B = 4096
H = 200
HP = 256
NC = 2
NS = 16
NW = NC * NS
RPW = B // NW
K_ROWS = 4
NGRP = RPW // K_ROWS


def _proj_body(x_ref, w_ref, b_ref, o_ref):
    x = x_ref[...]
    y = lax.dot_general(w_ref[...], x, (((1,), (1,)), ((), ())),
                        preferred_element_type=jnp.float32)
    o_ref[...] = (y + b_ref[0]).reshape(VB)


def _gather_body(proj_hbm, idx_hbm, out_hbm, idx_v, val_v, sem):
    wid = lax.axis_index("s") * NC + lax.axis_index("c")
    r0 = wid * RPW
    pltpu.sync_copy(idx_hbm.at[pl.ds(r0, RPW)], idx_v)

    def group(g, carry):
        base = g * K_ROWS
        copies = []
        for k in range(K_ROWS):
            r = base + k
            copies.append(pltpu.async_copy(
                proj_hbm.at[idx_v.at[r, pl.ds(0, 128)]],
                val_v.at[r, pl.ds(0, 128)], sem))
            copies.append(pltpu.async_copy(
                proj_hbm.at[idx_v.at[r, pl.ds(128, H - 128)]],
                val_v.at[r, pl.ds(128, H - 128)], sem))
        for c in copies:
            c.wait()
        return carry

    lax.fori_loop(0, NGRP, group, 0)
    pltpu.sync_copy(val_v, out_hbm.at[pl.ds(r0, RPW)])


def kernel(feature, lan_emb, W_w, b_w, W_out, b_out):
    proj = pl.pallas_call(
        _proj_body,
        grid=(NBLK,),
        in_specs=[
            pl.BlockSpec((VB, D), lambda i: (i, 0)),
            pl.BlockSpec((1, D), lambda i: (0, 0)),
            pl.BlockSpec(memory_space=pltpu.SMEM),
        ],
        out_specs=pl.BlockSpec((VB,), lambda i: (i,)),
        out_shape=jax.ShapeDtypeStruct((P,), jnp.float32),
    )(lan_emb, W_out, b_out)

    idx_pad = jnp.pad(feature.astype(jnp.int32), ((0, 0), (0, HP - H)))

    gather = functools.partial(
        pl.kernel,
        mesh=plsc.VectorSubcoreMesh(core_axis_name="c", subcore_axis_name="s"),
        out_type=jax.ShapeDtypeStruct((B, HP), jnp.float32),
        scratch_types=[
            pltpu.VMEM((RPW, HP), jnp.int32),
            pltpu.VMEM((RPW, HP), jnp.float32),
            pltpu.SemaphoreType.DMA,
        ],
    )(_gather_body)
    out_pad = gather(proj, idx_pad)

    return out_pad[:, :H]


# speedup vs baseline: 2.9475x; 2.9124x over previous
import functools

import jax
import jax.numpy as jnp
from jax import lax
from jax.experimental import pallas as pl
from jax.experimental.pallas import tpu as pltpu
from jax.experimental.pallas import tpu_sc as plsc

VOCAB = 1000000
D = 64
VB = 8192
NBLK = -(-VOCAB // VB)
SUBS = 8
P = NBLK * VB

B = 4096
H = 200
HP = 256
NC = 2
NS = 16
NW = NC * NS
RPW = B // NW
K_ROWS = 4
NGRP = RPW // K_ROWS


def _proj_body(xt_ref, w_ref, b_ref, o_ref):
    xt = xt_ref[...]
    y = lax.dot_general(w_ref[...], xt, (((1,), (0,)), ((), ())),
                        preferred_element_type=jnp.float32)
    o_ref[...] = (y + b_ref[0]).reshape(VB)


def _gather_body(proj_hbm, idx_hbm, out_hbm, idx_v, val_v, sem):
    wid = lax.axis_index("s") * NC + lax.axis_index("c")
    r0 = wid * RPW
    pltpu.sync_copy(idx_hbm.at[pl.ds(r0, RPW)], idx_v)

    def group(g, carry):
        base = g * K_ROWS
        copies = []
        for k in range(K_ROWS):
            r = base + k
            copies.append(pltpu.async_copy(
                proj_hbm.at[idx_v.at[r, pl.ds(0, 128)]],
                val_v.at[r, pl.ds(0, 128)], sem))
            copies.append(pltpu.async_copy(
                proj_hbm.at[idx_v.at[r, pl.ds(128, H - 128)]],
                val_v.at[r, pl.ds(128, H - 128)], sem))
        for c in copies:
            c.wait()
        return carry

    lax.fori_loop(0, NGRP, group, 0)
    pltpu.sync_copy(val_v, out_hbm.at[pl.ds(r0, RPW)])


def kernel(feature, lan_emb, W_w, b_w, W_out, b_out):
    proj = pl.pallas_call(
        _proj_body,
        grid=(NBLK,),
        in_specs=[
            pl.BlockSpec((D, VB), lambda i: (0, i)),
            pl.BlockSpec((1, D), lambda i: (0, 0)),
            pl.BlockSpec(memory_space=pltpu.SMEM),
        ],
        out_specs=pl.BlockSpec((VB,), lambda i: (i,)),
        out_shape=jax.ShapeDtypeStruct((P,), jnp.float32),
    )(lan_emb.T, W_out, b_out)

    idx_pad = jnp.pad(feature.astype(jnp.int32), ((0, 0), (0, HP - H)))

    gather = functools.partial(
        pl.kernel,
        mesh=plsc.VectorSubcoreMesh(core_axis_name="c", subcore_axis_name="s"),
        out_type=jax.ShapeDtypeStruct((B, HP), jnp.float32),
        scratch_types=[
            pltpu.VMEM((RPW, HP), jnp.int32),
            pltpu.VMEM((RPW, HP), jnp.float32),
            pltpu.SemaphoreType.DMA,
        ],
    )(_gather_body)
    out_pad = gather(proj, idx_pad)

    return out_pad[:, :H]


# speedup vs baseline: 3.2019x; 1.0863x over previous
import functools

import jax
import jax.numpy as jnp
from jax import lax
from jax.experimental import pallas as pl
from jax.experimental.pallas import tpu as pltpu
from jax.experimental.pallas import tpu_sc as plsc

VOCAB = 1000000
D = 64
VB = 8192
NBLK = -(-VOCAB // VB)
SUBS = 8
P = NBLK * VB

B = 4096
H = 200
NC = 2
NS = 16
NW = NC * NS
CPW = B // NW
K_CH = 8
NGRP = H // K_CH


def _proj_body(xt_ref, w_ref, b_ref, o_ref):
    xt = xt_ref[...]
    y = lax.dot_general(w_ref[...], xt, (((1,), (0,)), ((), ())),
                        preferred_element_type=jnp.float32)
    o_ref[...] = (y + b_ref[0]).reshape(VB)


def _gather_body(proj_hbm, idxt_hbm, outt_hbm, idx_v, val_v, sem):
    wid = lax.axis_index("s") * NC + lax.axis_index("c")
    c0 = wid * CPW
    pltpu.sync_copy(idxt_hbm.at[:, pl.ds(c0, CPW)], idx_v)

    def group(g, carry):
        base = g * K_CH
        copies = [pltpu.async_copy(
            proj_hbm.at[idx_v.at[base + k]],
            val_v.at[base + k], sem) for k in range(K_CH)]
        for c in copies:
            c.wait()
        return carry

    lax.fori_loop(0, NGRP, group, 0)
    pltpu.sync_copy(val_v, outt_hbm.at[:, pl.ds(c0, CPW)])


def kernel(feature, lan_emb, W_w, b_w, W_out, b_out):
    proj = pl.pallas_call(
        _proj_body,
        grid=(NBLK,),
        in_specs=[
            pl.BlockSpec((D, VB), lambda i: (0, i)),
            pl.BlockSpec((1, D), lambda i: (0, 0)),
            pl.BlockSpec(memory_space=pltpu.SMEM),
        ],
        out_specs=pl.BlockSpec((VB,), lambda i: (i,)),
        out_shape=jax.ShapeDtypeStruct((P,), jnp.float32),
    )(lan_emb.T, W_out, b_out)

    idxt = feature.astype(jnp.int32).T

    gather = functools.partial(
        pl.kernel,
        mesh=plsc.VectorSubcoreMesh(core_axis_name="c", subcore_axis_name="s"),
        out_type=jax.ShapeDtypeStruct((H, B), jnp.float32),
        scratch_types=[
            pltpu.VMEM((H, CPW), jnp.int32),
            pltpu.VMEM((H, CPW), jnp.float32),
            pltpu.SemaphoreType.DMA,
        ],
    )(_gather_body)
    outt = gather(proj, idxt)

    return outt.T


# speedup vs baseline: 3.9445x; 1.2319x over previous
import functools

import jax
import jax.numpy as jnp
from jax import lax
from jax.experimental import pallas as pl
from jax.experimental.pallas import tpu as pltpu
from jax.experimental.pallas import tpu_sc as plsc

VOCAB = 1000000
D = 64
VB = 16384
NBLK = -(-VOCAB // VB)
SUBS = 8
P = NBLK * VB

B = 4096
H = 200
NC = 2
NS = 16
NW = NC * NS
CPW = B // NW
K_CH = 8
NGRP = H // K_CH


def _proj_body(xt_ref, w_ref, b_ref, o_ref):
    xt = xt_ref[...]
    y = lax.dot_general(w_ref[...], xt, (((1,), (0,)), ((), ())),
                        preferred_element_type=jnp.float32)
    o_ref[...] = (y + b_ref[0]).reshape(VB)


def _gather_body(proj_hbm, idxt_hbm, outt_hbm, idx_v, val_v, sem):
    wid = lax.axis_index("s") * NC + lax.axis_index("c")
    c0 = wid * CPW
    pltpu.sync_copy(idxt_hbm.at[:, pl.ds(c0, CPW)], idx_v)

    def group(g, carry):
        base = g * K_CH
        copies = [pltpu.async_copy(
            proj_hbm.at[idx_v.at[base + k]],
            val_v.at[base + k], sem) for k in range(K_CH)]
        for c in copies:
            c.wait()
        return carry

    lax.fori_loop(0, NGRP, group, 0)
    pltpu.sync_copy(val_v, outt_hbm.at[:, pl.ds(c0, CPW)])


def kernel(feature, lan_emb, W_w, b_w, W_out, b_out):
    proj = pl.pallas_call(
        _proj_body,
        grid=(NBLK,),
        in_specs=[
            pl.BlockSpec((D, VB), lambda i: (0, i)),
            pl.BlockSpec((1, D), lambda i: (0, 0)),
            pl.BlockSpec(memory_space=pltpu.SMEM),
        ],
        out_specs=pl.BlockSpec((VB,), lambda i: (i,)),
        out_shape=jax.ShapeDtypeStruct((P,), jnp.float32),
    )(lan_emb.T, W_out, b_out)

    idxt = feature.astype(jnp.int32).T

    gather = functools.partial(
        pl.kernel,
        mesh=plsc.VectorSubcoreMesh(core_axis_name="c", subcore_axis_name="s"),
        out_type=jax.ShapeDtypeStruct((H, B), jnp.float32),
        scratch_types=[
            pltpu.VMEM((H, CPW), jnp.int32),
            pltpu.VMEM((H, CPW), jnp.float32),
            pltpu.SemaphoreType.DMA,
        ],
    )(_gather_body)
    outt = gather(proj, idxt)

    return outt.T


# speedup vs baseline: 4.2109x; 1.0675x over previous
import functools

import jax
import jax.numpy as jnp
from jax import lax
from jax.experimental import pallas as pl
from jax.experimental.pallas import tpu as pltpu
from jax.experimental.pallas import tpu_sc as plsc

VOCAB = 1000000
D = 64
VB = 32768
NBLK = -(-VOCAB // VB)
SUBS = 8
P = NBLK * VB

B = 4096
H = 200
NC = 2
NS = 16
NW = NC * NS
CPW = B // NW
K_CH = 8
NGRP = H // K_CH


def _proj_body(xt_ref, w_ref, b_ref, o_ref):
    xt = xt_ref[...]
    y = lax.dot_general(w_ref[...], xt, (((1,), (0,)), ((), ())),
                        preferred_element_type=jnp.float32)
    o_ref[...] = (y + b_ref[0]).reshape(VB)


def _gather_body(proj_hbm, idxt_hbm, outt_hbm, idx_v, val_v, sem):
    wid = lax.axis_index("s") * NC + lax.axis_index("c")
    c0 = wid * CPW
    pltpu.sync_copy(idxt_hbm.at[:, pl.ds(c0, CPW)], idx_v)

    def group(g, carry):
        base = g * K_CH
        copies = [pltpu.async_copy(
            proj_hbm.at[idx_v.at[base + k]],
            val_v.at[base + k], sem) for k in range(K_CH)]
        for c in copies:
            c.wait()
        return carry

    lax.fori_loop(0, NGRP, group, 0)
    pltpu.sync_copy(val_v, outt_hbm.at[:, pl.ds(c0, CPW)])


def kernel(feature, lan_emb, W_w, b_w, W_out, b_out):
    proj = pl.pallas_call(
        _proj_body,
        grid=(NBLK,),
        in_specs=[
            pl.BlockSpec((D, VB), lambda i: (0, i)),
            pl.BlockSpec((1, D), lambda i: (0, 0)),
            pl.BlockSpec(memory_space=pltpu.SMEM),
        ],
        out_specs=pl.BlockSpec((VB,), lambda i: (i,)),
        out_shape=jax.ShapeDtypeStruct((P,), jnp.float32),
    )(lan_emb.T, W_out, b_out)

    idxt = feature.astype(jnp.int32).T

    gather = functools.partial(
        pl.kernel,
        mesh=plsc.VectorSubcoreMesh(core_axis_name="c", subcore_axis_name="s"),
        out_type=jax.ShapeDtypeStruct((H, B), jnp.float32),
        scratch_types=[
            pltpu.VMEM((H, CPW), jnp.int32),
            pltpu.VMEM((H, CPW), jnp.float32),
            pltpu.SemaphoreType.DMA,
        ],
    )(_gather_body)
    outt = gather(proj, idxt)

    return outt.T
